# baseline (speedup 1.0000x reference)
import functools

import jax
import jax.numpy as jnp
from jax import lax
from jax.experimental import pallas as pl
from jax.experimental.pallas import tpu as pltpu
from jax.experimental.pallas import tpu_sc as plsc

_N = 128
_P = 64
_B = 4096
_EPS = 1e-06


def _table_body(s_ref, g_ref, m_ref):
    s = s_ref[...]
    g = g_ref[...]
    row = lax.broadcasted_iota(jnp.int32, (_N, _N), 0)
    col = lax.broadcasted_iota(jnp.int32, (_N, _N), 1)
    offdiag = (row != col).astype(s.dtype)
    sz = s * offdiag[None, :, :]
    denom = jnp.maximum(jnp.sum(jnp.abs(sz), axis=-1, keepdims=True), _EPS)
    sp = jnp.maximum(g, 0.0) + jnp.log1p(jnp.exp(-jnp.abs(g))) + 1e-06
    sp = sp * (_N / jnp.maximum(jnp.sum(sp, axis=-1, keepdims=True), _EPS))
    m_ref[...] = (sz / denom) * sp[:, :, None]


def _build_table(S, G):
    return pl.pallas_call(
        _table_body,
        out_shape=jax.ShapeDtypeStruct((_P, _N, _N), jnp.float32),
    )(S, G)


_NSCS = 2
_BPS = _B // _NSCS
_IDXCH = 128
_NREF = _BPS // _IDXCH
_LAG = 128


def _scs_body(table_hbm, idx_hbm, out_hbm, idx_s, spt, semt, sem0):
    cid = lax.axis_index("c")
    base = cid * _BPS
    tcopy = pltpu.async_copy(table_hbm, spt, semt)

    def drain_one():
        pltpu.make_async_copy(spt.at[0], out_hbm.at[base], sem0).wait()

    def refill(r, carry):
        pltpu.sync_copy(idx_hbm.at[pl.ds(base + r * _IDXCH, _IDXCH)], idx_s)

        def body(j, carry2):
            i = r * _IDXCH + j
            pltpu.async_copy(spt.at[idx_s[j]], out_hbm.at[base + i], sem0)

            @pl.when(i >= _LAG)
            def _():
                drain_one()

            return carry2

        lax.fori_loop(0, _IDXCH, body, carry)
        return carry

    tcopy.wait()
    lax.fori_loop(0, _NREF, refill, 0)
    for _ in range(_LAG):
        drain_one()


@jax.jit
def _sc_gather(table, idx):
    mesh = plsc.ScalarSubcoreMesh(axis_name="c", num_cores=_NSCS)
    f = functools.partial(
        pl.kernel,
        mesh=mesh,
        out_type=jax.ShapeDtypeStruct((_B, _N, _N), jnp.float32),
        scratch_types=[
            pltpu.SMEM((_IDXCH,), jnp.int32),
            pltpu.VMEM_SHARED((_P, _N, _N), jnp.float32),
            pltpu.SemaphoreType.DMA,
            pltpu.SemaphoreType.DMA,
        ],
    )(_scs_body)
    return f(table, idx)


def kernel(phases, S, G):
    table = _build_table(S.astype(jnp.float32), G.astype(jnp.float32))
    return _sc_gather(table, phases.astype(jnp.int32))


# speedup vs baseline: 1.0011x; 1.0011x over previous
import functools

import jax
import jax.numpy as jnp
from jax import lax
from jax.experimental import pallas as pl
from jax.experimental.pallas import tpu as pltpu
from jax.experimental.pallas import tpu_sc as plsc

_N = 128
_P = 64
_B = 4096
_EPS = 1e-06


def _table_body(s_ref, g_ref, m_ref):
    s = s_ref[...]
    g = g_ref[...]
    row = lax.broadcasted_iota(jnp.int32, (_N, _N), 0)
    col = lax.broadcasted_iota(jnp.int32, (_N, _N), 1)
    offdiag = (row != col).astype(s.dtype)
    sz = s * offdiag[None, :, :]
    denom = jnp.maximum(jnp.sum(jnp.abs(sz), axis=-1, keepdims=True), _EPS)
    sp = jnp.maximum(g, 0.0) + jnp.log1p(jnp.exp(-jnp.abs(g))) + 1e-06
    sp = sp * (_N / jnp.maximum(jnp.sum(sp, axis=-1, keepdims=True), _EPS))
    m_ref[...] = (sz / denom) * sp[:, :, None]


def _build_table(S, G):
    return pl.pallas_call(
        _table_body,
        out_shape=jax.ShapeDtypeStruct((_P, _N, _N), jnp.float32),
    )(S, G)


_NSCS = 2
_BPS = _B // _NSCS
_IDXCH = 128
_NREF = _BPS // _IDXCH
_LAG = 64


def _scs_body(table_hbm, idx_hbm, out_hbm, idx_s, spt, semt, sem0):
    cid = lax.axis_index("c")
    base = cid * _BPS
    tcopy = pltpu.async_copy(table_hbm, spt, semt)

    def drain_one():
        pltpu.make_async_copy(spt.at[0], out_hbm.at[base], sem0).wait()

    def refill(r, carry):
        pltpu.sync_copy(idx_hbm.at[pl.ds(base + r * _IDXCH, _IDXCH)], idx_s)

        def body(j, carry2):
            i = r * _IDXCH + j
            pltpu.async_copy(spt.at[idx_s[j]], out_hbm.at[base + i], sem0)

            @pl.when(i >= _LAG)
            def _():
                drain_one()

            return carry2

        lax.fori_loop(0, _IDXCH, body, carry)
        return carry

    tcopy.wait()
    lax.fori_loop(0, _NREF, refill, 0)
    for _ in range(_LAG):
        drain_one()


@jax.jit
def _sc_gather(table, idx):
    mesh = plsc.ScalarSubcoreMesh(axis_name="c", num_cores=_NSCS)
    f = functools.partial(
        pl.kernel,
        mesh=mesh,
        out_type=jax.ShapeDtypeStruct((_B, _N, _N), jnp.float32),
        scratch_types=[
            pltpu.SMEM((_IDXCH,), jnp.int32),
            pltpu.VMEM_SHARED((_P, _N, _N), jnp.float32),
            pltpu.SemaphoreType.DMA,
            pltpu.SemaphoreType.DMA,
        ],
    )(_scs_body)
    return f(table, idx)


def kernel(phases, S, G):
    table = _build_table(S.astype(jnp.float32), G.astype(jnp.float32))
    return _sc_gather(table, phases.astype(jnp.int32))
